# baseline (speedup 1.0000x reference)
import functools

import jax
import jax.numpy as jnp
from jax import lax
from jax.experimental import pallas as pl
from jax.experimental.pallas import tpu as pltpu
from jax.experimental.pallas import tpu_sc as plsc

B, S = 4096, 50
SP = 56
D = 256
O = 64
TP = B * SP
NC, NS = 2, 16
NW = NC * NS
T_PER_W = TP // NW
CHUNK = 112
NCHUNK = T_PER_W // CHUNK


@functools.partial(
    pl.kernel,
    out_type=jax.ShapeDtypeStruct((TP, D), jnp.float32),
    mesh=plsc.VectorSubcoreMesh(core_axis_name="c", subcore_axis_name="s"),
    scratch_types=[
        pltpu.VMEM((T_PER_W,), jnp.int32),
        pltpu.VMEM((CHUNK, D), jnp.float32),
        pltpu.SemaphoreType.DMA,
    ],
)
def _sc_gather(table_hbm, idx_hbm, h_hbm, idx_v, buf, sem):
    wid = lax.axis_index("s") * NC + lax.axis_index("c")
    base = wid * T_PER_W
    pltpu.sync_copy(idx_hbm.at[pl.ds(base, T_PER_W)], idx_v)

    def body(i, carry):
        off = i * CHUNK
        pltpu.async_copy(
            table_hbm.at[idx_v.at[pl.ds(off, CHUNK)]], buf, sem
        ).wait()
        pltpu.sync_copy(buf, h_hbm.at[pl.ds(base + off, CHUNK)])
        return carry

    lax.fori_loop(0, NCHUNK, body, 0)


BLK_B = 64
BLK_T = BLK_B * SP


def _tc_matmul_body(h_ref, w_ref, b_ref, o_ref):
    acc = lax.dot_general(
        h_ref[...], w_ref[...], (((1,), (1,)), ((), ())),
        preferred_element_type=jnp.float32,
    ) + b_ref[...]
    o_ref[...] = acc.reshape(BLK_B, SP, O)


def _tc_matmul(h, W, b):
    return pl.pallas_call(
        _tc_matmul_body,
        grid=(B // BLK_B,),
        in_specs=[
            pl.BlockSpec((BLK_T, D), lambda i: (i, 0)),
            pl.BlockSpec((O, D), lambda i: (0, 0)),
            pl.BlockSpec((1, O), lambda i: (0, 0)),
        ],
        out_specs=pl.BlockSpec((BLK_B, SP, O), lambda i: (i, 0, 0)),
        out_shape=jax.ShapeDtypeStruct((B, S, O), jnp.float32),
    )(h, W, b.reshape(1, O))


def kernel(x, table, W, b):
    xp = jnp.pad(x.astype(jnp.int32), ((0, 0), (0, SP - S))).reshape(TP)
    h = _sc_gather(table, xp)
    return _tc_matmul(h, W, b)


# speedup vs baseline: 1.0010x; 1.0010x over previous
import functools

import jax
import jax.numpy as jnp
from jax import lax
from jax.experimental import pallas as pl
from jax.experimental.pallas import tpu as pltpu
from jax.experimental.pallas import tpu_sc as plsc

B, S = 4096, 50
SP = 56
D = 256
O = 64
TP = B * SP
NC, NS = 2, 16
NW = NC * NS
T_PER_W = TP // NW
CHUNK = 128
NCHUNK = T_PER_W // CHUNK


@functools.partial(
    pl.kernel,
    out_type=jax.ShapeDtypeStruct((TP, D), jnp.float32),
    mesh=plsc.VectorSubcoreMesh(core_axis_name="c", subcore_axis_name="s"),
    scratch_types=[
        pltpu.VMEM((T_PER_W,), jnp.int32),
        pltpu.VMEM((CHUNK, D), jnp.float32),
        pltpu.SemaphoreType.DMA,
    ],
)
def _sc_gather(table_hbm, idx_hbm, h_hbm, idx_v, buf, sem):
    wid = lax.axis_index("s") * NC + lax.axis_index("c")
    base = wid * T_PER_W
    pltpu.sync_copy(idx_hbm.at[pl.ds(base, T_PER_W)], idx_v)

    def body(i, carry):
        off = i * CHUNK
        pltpu.async_copy(
            table_hbm.at[idx_v.at[pl.ds(off, CHUNK)]], buf, sem
        ).wait()
        pltpu.sync_copy(buf, h_hbm.at[pl.ds(base + off, CHUNK)])
        return carry

    lax.fori_loop(0, NCHUNK, body, 0)


BLK_B = 64
BLK_T = BLK_B * SP


def _tc_matmul_body(h_ref, w_ref, b_ref, o_ref):
    acc = lax.dot_general(
        h_ref[...], w_ref[...], (((1,), (1,)), ((), ())),
        preferred_element_type=jnp.float32,
    ) + b_ref[...]
    o_ref[...] = acc.reshape(BLK_B, SP, O)


def _tc_matmul(h, W, b):
    return pl.pallas_call(
        _tc_matmul_body,
        grid=(B // BLK_B,),
        in_specs=[
            pl.BlockSpec((BLK_T, D), lambda i: (i, 0)),
            pl.BlockSpec((O, D), lambda i: (0, 0)),
            pl.BlockSpec((1, O), lambda i: (0, 0)),
        ],
        out_specs=pl.BlockSpec((BLK_B, SP, O), lambda i: (i, 0, 0)),
        out_shape=jax.ShapeDtypeStruct((B, S, O), jnp.float32),
    )(h, W, b.reshape(1, O))


def kernel(x, table, W, b):
    xp = jnp.pad(x.astype(jnp.int32), ((0, 0), (0, SP - S))).reshape(TP)
    h = _sc_gather(table, xp)
    return _tc_matmul(h, W, b)


# speedup vs baseline: 3.6268x; 3.6233x over previous
import functools

import jax
import jax.numpy as jnp
from jax import lax
from jax.experimental import pallas as pl
from jax.experimental.pallas import tpu as pltpu
from jax.experimental.pallas import tpu_sc as plsc

B, S = 4096, 50
SP = 56
D = 256
O = 64
TP = B * SP
NC, NS = 2, 16
NW = NC * NS
T_PER_W = TP // NW
CHUNK = 128
NCHUNK = T_PER_W // CHUNK


@functools.partial(
    pl.kernel,
    out_type=jax.ShapeDtypeStruct((TP, D), jnp.float32),
    mesh=plsc.VectorSubcoreMesh(core_axis_name="c", subcore_axis_name="s"),
    scratch_types=[
        pltpu.VMEM((T_PER_W,), jnp.int32),
        pltpu.VMEM((CHUNK, D), jnp.float32),
        pltpu.SemaphoreType.DMA,
    ],
)
def _sc_gather(table_hbm, idx_hbm, h_hbm, idx_v, buf, sem):
    wid = lax.axis_index("s") * NC + lax.axis_index("c")
    base = wid * T_PER_W
    pltpu.sync_copy(idx_hbm.at[pl.ds(base, T_PER_W)], idx_v)

    def body(i, carry):
        off = i * CHUNK
        pltpu.async_copy(
            table_hbm.at[idx_v.at[pl.ds(off, CHUNK)]], buf, sem
        ).wait()
        pltpu.sync_copy(buf, h_hbm.at[pl.ds(base + off, CHUNK)])
        return carry

    lax.fori_loop(0, NCHUNK, body, 0)


BLK_B = 64
BLK_T = BLK_B * SP


def _tc_matmul_body(h_ref, w_ref, b_ref, o_ref):
    acc = lax.dot_general(
        h_ref[...], w_ref[...], (((1,), (1,)), ((), ())),
        preferred_element_type=jnp.float32,
    ) + b_ref[...]
    o_ref[...] = acc.reshape(BLK_B, SP, O)


def _tc_matmul(h, W, b):
    return pl.pallas_call(
        _tc_matmul_body,
        grid=(B // BLK_B,),
        in_specs=[
            pl.BlockSpec((BLK_T, D), lambda i: (i, 0)),
            pl.BlockSpec((O, D), lambda i: (0, 0)),
            pl.BlockSpec((1, O), lambda i: (0, 0)),
        ],
        out_specs=pl.BlockSpec((BLK_B, SP, O), lambda i: (i, 0, 0)),
        out_shape=jax.ShapeDtypeStruct((B, S, O), jnp.float32),
    )(h, W, b.reshape(1, O))


def kernel(x, table, W, b):
    xp = jnp.pad(x.astype(jnp.int32), ((0, 0), (0, SP - S)),
                 mode="edge").reshape(TP)
    h = _sc_gather(table, xp)
    return _tc_matmul(h, W, b)


# speedup vs baseline: 3.9147x; 1.0794x over previous
import functools

import jax
import jax.numpy as jnp
from jax import lax
from jax.experimental import pallas as pl
from jax.experimental.pallas import tpu as pltpu
from jax.experimental.pallas import tpu_sc as plsc

B, S = 4096, 50
SP = 56
D = 256
O = 64
TP = B * SP
NC, NS = 2, 16
NW = NC * NS
T_PER_W = TP // NW
CHUNK = 128
NCHUNK = T_PER_W // CHUNK


@functools.partial(
    pl.kernel,
    out_type=jax.ShapeDtypeStruct((TP, D), jnp.float32),
    mesh=plsc.VectorSubcoreMesh(core_axis_name="c", subcore_axis_name="s"),
    scratch_types=[
        pltpu.VMEM((T_PER_W,), jnp.int32),
        pltpu.VMEM((CHUNK, D), jnp.float32),
        pltpu.SemaphoreType.DMA,
    ],
)
def _sc_gather(table_hbm, idx_hbm, h_hbm, idx_v, buf, sem):
    wid = lax.axis_index("s") * NC + lax.axis_index("c")
    base = wid * T_PER_W
    pltpu.sync_copy(idx_hbm.at[pl.ds(base, T_PER_W)], idx_v)

    def body(i, carry):
        off = i * CHUNK
        pltpu.async_copy(
            table_hbm.at[idx_v.at[pl.ds(off, CHUNK)]], buf, sem
        ).wait()
        pltpu.sync_copy(buf, h_hbm.at[pl.ds(base + off, CHUNK)])
        return carry

    lax.fori_loop(0, NCHUNK, body, 0)


BLK_B = 64
BLK_T = BLK_B * SP


def _tc_matmul_body(h_ref, w_ref, b_ref, o_ref):
    acc = lax.dot_general(
        h_ref[...], w_ref[...], (((1,), (1,)), ((), ())),
        preferred_element_type=jnp.float32,
    ) + b_ref[...]
    o_ref[...] = acc.reshape(BLK_B, SP, O)


def _tc_matmul(h, W, b):
    return pl.pallas_call(
        _tc_matmul_body,
        grid=(B // BLK_B,),
        in_specs=[
            pl.BlockSpec((BLK_T, D), lambda i: (i, 0)),
            pl.BlockSpec((O, D), lambda i: (0, 0)),
            pl.BlockSpec((1, O), lambda i: (0, 0)),
        ],
        out_specs=pl.BlockSpec((BLK_B, SP, O), lambda i: (i, 0, 0)),
        out_shape=jax.ShapeDtypeStruct((B, SP, O), jnp.float32),
    )(h, W, b.reshape(1, O))


def kernel(x, table, W, b):
    xp = jnp.pad(x.astype(jnp.int32), ((0, 0), (0, SP - S)),
                 mode="edge").reshape(TP)
    h = _sc_gather(table, xp)
    return _tc_matmul(h, W, b)[:, :S, :]
